# Initial kernel scaffold; baseline (speedup 1.0000x reference)
import jax
import jax.numpy as jnp
from jax.experimental import pallas as pl


def kernel(hidden_states, gate_w, w1, w3, w2):
    raise NotImplementedError("write your pallas kernel here")


# speedup vs baseline: 1.5438x; 1.5438x over previous
import jax
import jax.numpy as jnp
from jax.experimental import pallas as pl
from jax.experimental.pallas import tpu as pltpu

F_TILE = 512


def _router_body(x_ref, gw_ref, logits_ref, wd_ref):
    x = x_ref[...]
    gw = gw_ref[...]
    logits = jax.lax.dot_general(x, gw, (((1,), (1,)), ((), ())),
                                 preferred_element_type=jnp.float32)
    logits_ref[...] = logits
    ne = logits.shape[1]
    col = jax.lax.broadcasted_iota(jnp.int32, logits.shape, 1)
    m1 = jnp.max(logits, axis=1, keepdims=True)
    e0 = jnp.min(jnp.where(logits == m1, col, ne), axis=1, keepdims=True)
    masked = jnp.where(col == e0, jnp.float32(-1e30), logits)
    m2 = jnp.max(masked, axis=1, keepdims=True)
    e1 = jnp.min(jnp.where(masked == m2, col, ne), axis=1, keepdims=True)
    sel = (col == e0) | (col == e1)
    denom = 1.0 + jnp.exp(m2 - m1)
    wd_ref[...] = jnp.where(sel, jnp.exp(logits - m1) / denom, jnp.float32(0.0))


def _dense_body(x_ref, wd_ref, w1_ref, w3_ref, w2_ref, out_ref, xbf_ref):
    e = pl.program_id(0)
    f = pl.program_id(1)

    @pl.when((e == 0) & (f == 0))
    def _init():
        xbf_ref[...] = x_ref[...].astype(jnp.bfloat16)
        out_ref[...] = jnp.zeros_like(out_ref)

    xb = xbf_ref[...]
    w1b = w1_ref[0].astype(jnp.bfloat16)
    w3b = w3_ref[0].astype(jnp.bfloat16)
    w2b = w2_ref[0].astype(jnp.bfloat16)
    y1 = jax.lax.dot_general(xb, w1b, (((1,), (1,)), ((), ())),
                             preferred_element_type=jnp.float32)
    y3 = jax.lax.dot_general(xb, w3b, (((1,), (1,)), ((), ())),
                             preferred_element_type=jnp.float32)
    h = (y1 * jax.lax.logistic(y1)) * y3
    wd = wd_ref[...]
    col = jax.lax.broadcasted_iota(jnp.int32, wd.shape, 1)
    wcol = jnp.sum(jnp.where(col == e, wd, jnp.float32(0.0)), axis=1,
                   keepdims=True)
    hw = (h * wcol).astype(jnp.bfloat16)
    out_ref[...] += jax.lax.dot_general(hw, w2b, (((1,), (1,)), ((), ())),
                                        preferred_element_type=jnp.float32)


def kernel(hidden_states, gate_w, w1, w3, w2):
    b, s, hd = hidden_states.shape
    ne, ffn, _ = w1.shape
    t = b * s
    x2 = hidden_states.reshape(t, hd)

    logits, wd = pl.pallas_call(
        _router_body,
        out_shape=(jax.ShapeDtypeStruct((t, ne), jnp.float32),
                   jax.ShapeDtypeStruct((t, ne), jnp.float32)),
    )(x2, gate_w)

    nf = ffn // F_TILE
    out = pl.pallas_call(
        _dense_body,
        grid=(ne, nf),
        in_specs=[
            pl.BlockSpec((t, hd), lambda e, f: (0, 0)),
            pl.BlockSpec((t, ne), lambda e, f: (0, 0)),
            pl.BlockSpec((1, F_TILE, hd), lambda e, f: (e, f, 0)),
            pl.BlockSpec((1, F_TILE, hd), lambda e, f: (e, f, 0)),
            pl.BlockSpec((1, hd, F_TILE), lambda e, f: (e, 0, f)),
        ],
        out_specs=pl.BlockSpec((t, hd), lambda e, f: (0, 0)),
        out_shape=jax.ShapeDtypeStruct((t, hd), jnp.float32),
        scratch_shapes=[pltpu.VMEM((t, hd), jnp.bfloat16)],
    )(x2, wd, w1, w3, w2)

    return out.reshape(b, s, hd), logits
